# Initial kernel scaffold; baseline (speedup 1.0000x reference)
import jax
import jax.numpy as jnp
from jax.experimental import pallas as pl


def kernel(x, current_frame_idx, rel_table, temp_table):
    raise NotImplementedError("write your pallas kernel here")


# speedup vs baseline: 12.0528x; 12.0528x over previous
import jax
import jax.numpy as jnp
from jax import lax
from jax.experimental import pallas as pl
from jax.experimental.pallas import tpu as pltpu


def _body(cur_ref, x_ref, rel_ref, temp_ref, o_ref, *, s_blk, seq_len):
    i = pl.program_id(0)
    s0 = i * s_blk

    padn = rel_ref.shape[0]
    k_io = lax.broadcasted_iota(jnp.int32, (s_blk, padn), 1)
    s_io = lax.broadcasted_iota(jnp.int32, (s_blk, padn), 0) + s0
    w = ((k_io >= s_io) & (k_io <= s_io + (seq_len - 1))).astype(jnp.float32)
    rel_sum = jnp.dot(w, rel_ref[...], preferred_element_type=jnp.float32)

    batch = x_ref.shape[1]
    cur = cur_ref[0, :][None, :, None]
    s_ids = lax.broadcasted_iota(jnp.int32, (s_blk, batch, 1), 0) + s0
    t0 = temp_ref[0, :][None, None, :]
    t1 = temp_ref[1, :][None, None, :]
    t2 = temp_ref[2, :][None, None, :]
    temporal = jnp.where(s_ids < cur, t0, jnp.where(s_ids == cur, t1, t2))

    o_ref[...] = x_ref[...] + temporal + rel_sum[:, None, :]


def kernel(x, current_frame_idx, rel_table, temp_table):
    seq_len, batch, embed = x.shape
    s_blk = 20
    grid = seq_len // s_blk

    padn = (rel_table.shape[0] + 7) // 8 * 8
    rel_padded = jnp.zeros((padn, embed), rel_table.dtype).at[: rel_table.shape[0]].set(rel_table)
    cur2 = current_frame_idx.astype(jnp.int32).reshape(1, batch)

    import functools

    return pl.pallas_call(
        functools.partial(_body, s_blk=s_blk, seq_len=seq_len),
        grid=(grid,),
        in_specs=[
            pl.BlockSpec((1, batch), lambda i: (0, 0)),
            pl.BlockSpec((s_blk, batch, embed), lambda i: (i, 0, 0)),
            pl.BlockSpec((padn, embed), lambda i: (0, 0)),
            pl.BlockSpec((3, embed), lambda i: (0, 0)),
        ],
        out_specs=pl.BlockSpec((s_blk, batch, embed), lambda i: (i, 0, 0)),
        out_shape=jax.ShapeDtypeStruct((seq_len, batch, embed), x.dtype),
    )(cur2, x, rel_padded, temp_table)


# speedup vs baseline: 12.2905x; 1.0197x over previous
import jax
import jax.numpy as jnp
from jax import lax
from jax.experimental import pallas as pl
from jax.experimental.pallas import tpu as pltpu


def _body(cur_ref, x_ref, rel_ref, temp_ref, o_ref, *, s_blk, seq_len):
    i = pl.program_id(0)
    s0 = i * s_blk

    padn = rel_ref.shape[0]
    k_io = lax.broadcasted_iota(jnp.int32, (s_blk, padn), 1)
    s_io = lax.broadcasted_iota(jnp.int32, (s_blk, padn), 0) + s0
    w = ((k_io >= s_io) & (k_io <= s_io + (seq_len - 1))).astype(jnp.float32)
    rel_sum = jnp.dot(w, rel_ref[...], preferred_element_type=jnp.float32)

    batch = x_ref.shape[1]
    cur = cur_ref[0, :][None, :, None]
    s_ids = lax.broadcasted_iota(jnp.int32, (s_blk, batch, 1), 0) + s0
    t0 = temp_ref[0, :][None, None, :]
    t1 = temp_ref[1, :][None, None, :]
    t2 = temp_ref[2, :][None, None, :]
    temporal = jnp.where(s_ids < cur, t0, jnp.where(s_ids == cur, t1, t2))

    o_ref[...] = x_ref[...] + temporal + rel_sum[:, None, :]


def kernel(x, current_frame_idx, rel_table, temp_table):
    seq_len, batch, embed = x.shape
    s_blk = 10
    grid = seq_len // s_blk

    padn = (rel_table.shape[0] + 7) // 8 * 8
    rel_padded = jnp.zeros((padn, embed), rel_table.dtype).at[: rel_table.shape[0]].set(rel_table)
    cur2 = current_frame_idx.astype(jnp.int32).reshape(1, batch)

    import functools

    return pl.pallas_call(
        functools.partial(_body, s_blk=s_blk, seq_len=seq_len),
        grid=(grid,),
        in_specs=[
            pl.BlockSpec((1, batch), lambda i: (0, 0)),
            pl.BlockSpec((s_blk, batch, embed), lambda i: (i, 0, 0)),
            pl.BlockSpec((padn, embed), lambda i: (0, 0)),
            pl.BlockSpec((3, embed), lambda i: (0, 0)),
        ],
        out_specs=pl.BlockSpec((s_blk, batch, embed), lambda i: (i, 0, 0)),
        out_shape=jax.ShapeDtypeStruct((seq_len, batch, embed), x.dtype),
    )(cur2, x, rel_padded, temp_table)
